# Initial kernel scaffold; baseline (speedup 1.0000x reference)
import functools

import jax
import jax.numpy as jnp
from jax import lax
from jax.experimental import pallas as pl
from jax.experimental.pallas import tpu as pltpu
from jax.experimental.pallas import tpu_sc as plsc

B = 1024
L = 50
V = 1000
LANES = 16

_info = plsc.get_sparse_core_info()
_NC = _info.num_cores
_NS = _info.num_subcores
_NW = _NC * _NS
_ROWS_PER_W = B // _NW

_ZERO_CHUNKS = V // LANES
_ZERO_TAIL = V - LANES
_FULL_IDX_CHUNKS = L // LANES
_IDX_TAIL = L - LANES
_TAIL_LANES = L - _FULL_IDX_CHUNKS * LANES


def _sc_kernel(x_hbm, out_hbm, idx_v, acc_v):
    wid = lax.axis_index("s") * _NC + lax.axis_index("c")
    base = wid * _ROWS_PER_W
    pltpu.sync_copy(x_hbm.at[pl.ds(base, _ROWS_PER_W)], idx_v)

    ones = jnp.ones((LANES,), jnp.float32)
    zeros = jnp.zeros((LANES,), jnp.float32)
    tail_mask = lax.iota(jnp.int32, LANES) >= (LANES - _TAIL_LANES)

    def row_body(r, carry):
        def zero_body(c, carry2):
            acc_v[r, pl.ds(c * LANES, LANES)] = zeros
            return carry2

        lax.fori_loop(0, _ZERO_CHUNKS, zero_body, 0)
        acc_v[r, pl.ds(_ZERO_TAIL, LANES)] = zeros

        rvec = jnp.full((LANES,), r, jnp.int32)
        for c in range(_FULL_IDX_CHUNKS):
            col = idx_v[r, pl.ds(c * LANES, LANES)]
            plsc.addupdate_scatter(acc_v, [rvec, col], ones)
        col = idx_v[r, pl.ds(_IDX_TAIL, LANES)]
        plsc.addupdate_scatter(acc_v, [rvec, col], ones, mask=tail_mask)
        return carry

    lax.fori_loop(0, _ROWS_PER_W, row_body, 0)
    pltpu.sync_copy(acc_v, out_hbm.at[pl.ds(base, _ROWS_PER_W)])


@jax.jit
def kernel(x):
    mesh = plsc.VectorSubcoreMesh(core_axis_name="c", subcore_axis_name="s")
    run = functools.partial(
        pl.kernel,
        mesh=mesh,
        out_type=jax.ShapeDtypeStruct((B, V), jnp.float32),
        scratch_types=[
            pltpu.VMEM((_ROWS_PER_W, L), jnp.int32),
            pltpu.VMEM((_ROWS_PER_W, V), jnp.float32),
        ],
    )(_sc_kernel)
    return run(x.astype(jnp.int32))


# speedup vs baseline: 1.0700x; 1.0700x over previous
import functools

import jax
import jax.numpy as jnp
from jax import lax
from jax.experimental import pallas as pl
from jax.experimental.pallas import tpu as pltpu
from jax.experimental.pallas import tpu_sc as plsc

B = 1024
L = 50
V = 1000
LANES = 16

_NC = 2
_NS = 16
_NW = _NC * _NS
_ROWS_PER_W = B // _NW

_ZERO_CHUNKS = V // LANES
_ZERO_TAIL = V - LANES
_FULL_IDX_CHUNKS = L // LANES
_IDX_TAIL = L - LANES
_TAIL_LANES = L - _FULL_IDX_CHUNKS * LANES


def _sc_kernel(x_hbm, out_hbm, idx_v, acc_v):
    wid = lax.axis_index("s") * _NC + lax.axis_index("c")
    pltpu.sync_copy(x_hbm.at[pl.ds(wid * (_ROWS_PER_W * L), _ROWS_PER_W * L)],
                    idx_v)

    ones = jnp.ones((LANES,), jnp.float32)
    zeros = jnp.zeros((LANES,), jnp.float32)
    tail_mask = lax.iota(jnp.int32, LANES) >= (LANES - _TAIL_LANES)

    def row_body(r, carry):
        acc_base = r * V
        idx_base = r * L

        def zero_body(c, carry2):
            acc_v[pl.ds(acc_base + c * LANES, LANES)] = zeros
            return carry2

        lax.fori_loop(0, _ZERO_CHUNKS, zero_body, 0)
        acc_v[pl.ds(acc_base + _ZERO_TAIL, LANES)] = zeros

        for c in range(_FULL_IDX_CHUNKS):
            col = idx_v[pl.ds(idx_base + c * LANES, LANES)]
            plsc.addupdate_scatter(acc_v, [acc_base + col], ones)
        col = idx_v[pl.ds(idx_base + _IDX_TAIL, LANES)]
        plsc.addupdate_scatter(acc_v, [acc_base + col], ones, mask=tail_mask)
        return carry

    lax.fori_loop(0, _ROWS_PER_W, row_body, 0)
    pltpu.sync_copy(acc_v,
                    out_hbm.at[pl.ds(wid * (_ROWS_PER_W * V), _ROWS_PER_W * V)])


@jax.jit
def kernel(x):
    mesh = plsc.VectorSubcoreMesh(core_axis_name="c", subcore_axis_name="s")
    run = functools.partial(
        pl.kernel,
        mesh=mesh,
        compiler_params=pltpu.CompilerParams(
            use_tc_tiling_on_sc=False,
            needs_layout_passes=False,
        ),
        out_type=jax.ShapeDtypeStruct((B * V,), jnp.float32),
        scratch_types=[
            pltpu.VMEM((_ROWS_PER_W * L,), jnp.int32),
            pltpu.VMEM((_ROWS_PER_W * V,), jnp.float32),
        ],
    )(_sc_kernel)
    return run(x.astype(jnp.int32).reshape(B * L)).reshape(B, V)


# speedup vs baseline: 1.3567x; 1.2679x over previous
import functools

import jax
import jax.numpy as jnp
from jax import lax
from jax.experimental import pallas as pl
from jax.experimental.pallas import tpu as pltpu
from jax.experimental.pallas import tpu_sc as plsc

B = 1024
L = 50
V = 1000
LANES = 16

_NC = 2
_NS = 16
_NW = _NC * _NS
_ROWS_PER_W = B // _NW

_ZERO_CHUNKS = V // LANES
_ZERO_TAIL = V - LANES
_FULL_IDX_CHUNKS = L // LANES
_IDX_TAIL = L - LANES
_TAIL_LANES = L - _FULL_IDX_CHUNKS * LANES


def _sc_kernel(x_hbm, out_hbm, idx_v, acc_v):
    wid = lax.axis_index("s") * _NC + lax.axis_index("c")
    pltpu.sync_copy(x_hbm.at[pl.ds(wid * (_ROWS_PER_W * L), _ROWS_PER_W * L)],
                    idx_v)

    ones = jnp.ones((LANES,), jnp.float32)
    zeros = jnp.zeros((LANES,), jnp.float32)
    tail_mask = lax.iota(jnp.int32, LANES) >= (LANES - _TAIL_LANES)

    @plsc.parallel_loop(0, _ROWS_PER_W * V, step=LANES, unroll=16)
    def _zero(i):
        acc_v[pl.ds(i, LANES)] = zeros

    @plsc.parallel_loop(0, _ROWS_PER_W, step=1, unroll=4)
    def _scatter(r):
        acc_base = r * V
        idx_base = r * L
        for c in range(_FULL_IDX_CHUNKS):
            col = idx_v[pl.ds(idx_base + c * LANES, LANES)]
            plsc.addupdate_scatter(acc_v, [acc_base + col], ones)
        col = idx_v[pl.ds(idx_base + _IDX_TAIL, LANES)]
        plsc.addupdate_scatter(acc_v, [acc_base + col], ones, mask=tail_mask)
    pltpu.sync_copy(acc_v,
                    out_hbm.at[pl.ds(wid * (_ROWS_PER_W * V), _ROWS_PER_W * V)])


@jax.jit
def kernel(x):
    mesh = plsc.VectorSubcoreMesh(core_axis_name="c", subcore_axis_name="s")
    run = functools.partial(
        pl.kernel,
        mesh=mesh,
        compiler_params=pltpu.CompilerParams(
            use_tc_tiling_on_sc=False,
            needs_layout_passes=False,
        ),
        out_type=jax.ShapeDtypeStruct((B * V,), jnp.float32),
        scratch_types=[
            pltpu.VMEM((_ROWS_PER_W * L,), jnp.int32),
            pltpu.VMEM((_ROWS_PER_W * V,), jnp.float32),
        ],
    )(_sc_kernel)
    return run(x.astype(jnp.int32).reshape(B * L)).reshape(B, V)
